# baseline (speedup 1.0000x reference)
import jax
import jax.numpy as jnp
from jax.experimental import pallas as pl
from jax.experimental.pallas import tpu as pltpu

_E = 160000
_IN = 128
_OUT = 128
_H = 8
_DK = 16
_NE = 6
_NT = 4
_ML = 240

_B = 1280
_G = _E // _B

_bf16 = jnp.bfloat16
_f32 = jnp.float32


def _body(dt_ref, tau_ref, et_ref, hs_ref, qt_ref,
          emb_ref, wrte_t_ref, brte_ref, wkv_ref, bk_ref, bv_ref,
          bdatt_ref, bdmsg_ref, s_ref, mu_ref,
          att_ref, m_ref):
    temb = jnp.dot(emb_ref[...].astype(_bf16), wrte_t_ref[...],
                   preferred_element_type=_f32) + brte_ref[...]

    dt = dt_ref[0]
    tau = tau_ref[0]
    et = et_ref[0]

    iota_ml = jax.lax.broadcasted_iota(jnp.int32, (_B, _ML), 1)
    oh_dt = (iota_ml == dt).astype(_bf16)
    hhat = hs_ref[...] + jnp.dot(oh_dt, temb.astype(_bf16),
                                 preferred_element_type=_f32)

    kv = jnp.dot(hhat.astype(_bf16), wkv_ref[...],
                 preferred_element_type=_f32).astype(_bf16)

    def sel(x, idx, n, base):
        y = x[:, base * _OUT:(base + 1) * _OUT]
        for t in range(1, n):
            y = jnp.where(idx == t, x[:, (base + t) * _OUT:(base + t + 1) * _OUT], y)
        return y

    k = sel(kv, tau, _NT, 0)
    v = sel(kv, tau, _NT, _NT)

    def rowsel(table_ref, idx, n):
        y = table_ref[0, :]
        for t in range(1, n):
            y = jnp.where(idx == t, table_ref[t, :], y)
        return y

    k = k + rowsel(bk_ref, tau, _NT).astype(_bf16)
    v = v + rowsel(bv_ref, tau, _NT).astype(_bf16)

    attk6 = jnp.dot(k, bdatt_ref[...],
                    preferred_element_type=_f32).astype(_bf16)
    msg6 = jnp.dot(v, bdmsg_ref[...],
                   preferred_element_type=_f32).astype(_bf16)

    attk = sel(attk6, et, _NE, 0)
    msg = sel(msg6, et, _NE, 0)
    m_ref[...] = msg.astype(_f32).reshape(_B, _H, _DK)

    prod = qt_ref[...].reshape(_B, _IN).astype(_bf16) * attk
    att8 = jnp.dot(prod, s_ref[...], preferred_element_type=_f32)
    mu_e = rowsel(mu_ref, et, _NE)
    att_ref[...] = att8 * mu_e * (1.0 / (_DK ** 0.5))


def kernel(h_s, Q_t, etype, tau_s, tau_t, dt, emb, W_rte, b_rte,
           W_K, b_K, W_V, b_V, W_att, W_msg, mu):
    del tau_t

    wrte_t = W_rte.T.astype(_bf16)
    wkv = jnp.concatenate([W_K, W_V], axis=0)
    wkv = jnp.transpose(wkv, (2, 0, 1)).reshape(_IN, 2 * _NT * _OUT)
    wkv = wkv.astype(_bf16)
    def _bd(w):
        b = jnp.zeros((_NE, _OUT, _OUT), _f32)
        wt = jnp.transpose(w, (0, 2, 1))
        for h in range(_H):
            b = b.at[:, h * _DK:(h + 1) * _DK, h * _DK:(h + 1) * _DK].set(wt)
        return jnp.transpose(b, (1, 0, 2)).reshape(_OUT, _NE * _OUT).astype(_bf16)
    bdatt = _bd(W_att)
    bdmsg = _bd(W_msg)
    seg = (jax.lax.broadcasted_iota(jnp.int32, (_OUT, _H), 0) // _DK ==
           jax.lax.broadcasted_iota(jnp.int32, (_OUT, _H), 1)).astype(_bf16)

    dt3 = dt.astype(jnp.int32).reshape(_G, _B, 1)
    tau3 = tau_s.astype(jnp.int32).reshape(_G, _B, 1)
    et3 = etype.astype(jnp.int32).reshape(_G, _B, 1)

    idx_spec = pl.BlockSpec((1, _B, 1), lambda i: (i, 0, 0))
    row_spec = pl.BlockSpec((_B, _IN), lambda i: (i, 0))
    hd_spec = pl.BlockSpec((_B, _H, _DK), lambda i: (i, 0, 0))

    def w_spec(shape):
        return pl.BlockSpec(shape, lambda i: tuple(0 for _ in shape))

    att, m = pl.pallas_call(
        _body,
        grid=(_G,),
        in_specs=[idx_spec, idx_spec, idx_spec, row_spec, hd_spec,
                  w_spec((_ML, _IN)), w_spec((_IN, _IN)), w_spec((_IN,)),
                  w_spec((_IN, 2 * _NT * _OUT)),
                  w_spec((_NT, _OUT)), w_spec((_NT, _OUT)),
                  w_spec((_OUT, _NE * _OUT)), w_spec((_OUT, _NE * _OUT)),
                  w_spec((_OUT, _H)), w_spec((_NE, _H))],
        out_specs=[pl.BlockSpec((_B, _H), lambda i: (i, 0)),
                   hd_spec],
        out_shape=[jax.ShapeDtypeStruct((_E, _H), _f32),
                   jax.ShapeDtypeStruct((_E, _H, _DK), _f32)],
    )(dt3, tau3, et3, h_s, Q_t,
      emb, wrte_t, b_rte, wkv, b_K, b_V, bdatt, bdmsg, seg, mu)

    return att, m


# speedup vs baseline: 1.8496x; 1.8496x over previous
import jax
import jax.numpy as jnp
from jax.experimental import pallas as pl
from jax.experimental.pallas import tpu as pltpu

_E = 160000
_IN = 128
_OUT = 128
_H = 8
_DK = 16
_NE = 6
_NT = 4
_ML = 240

_B = 3200
_G = _E // _B

_bf16 = jnp.bfloat16
_f32 = jnp.float32


def _body(dt_ref, tau_ref, et_ref, hs_ref, qt_ref,
          emb_ref, wrte_t_ref, brte_ref, wkv_ref, bk_ref, bv_ref,
          bdatt_ref, bdmsg_ref, s_ref, mu_ref,
          att_ref, m_ref):
    temb = jnp.dot(emb_ref[...].astype(_bf16), wrte_t_ref[...],
                   preferred_element_type=_f32) + brte_ref[...]

    dt = dt_ref[0]
    tau = tau_ref[0]
    et = et_ref[0]

    iota_ml = jax.lax.broadcasted_iota(jnp.int32, (_B, _ML), 1)
    oh_dt = (iota_ml == dt).astype(_bf16)
    hhat = hs_ref[...] + jnp.dot(oh_dt, temb.astype(_bf16),
                                 preferred_element_type=_f32)

    kv = jnp.dot(hhat.astype(_bf16), wkv_ref[...],
                 preferred_element_type=_f32).astype(_bf16)

    def sel(x, idx, n, base):
        y = x[:, base * _OUT:(base + 1) * _OUT]
        for t in range(1, n):
            y = jnp.where(idx == t, x[:, (base + t) * _OUT:(base + t + 1) * _OUT], y)
        return y

    k = sel(kv, tau, _NT, 0)
    v = sel(kv, tau, _NT, _NT)

    def rowsel(table_ref, idx, n):
        y = table_ref[0, :]
        for t in range(1, n):
            y = jnp.where(idx == t, table_ref[t, :], y)
        return y

    k = k + rowsel(bk_ref, tau, _NT).astype(_bf16)
    v = v + rowsel(bv_ref, tau, _NT).astype(_bf16)

    attk6 = jnp.dot(k, bdatt_ref[...],
                    preferred_element_type=_f32).astype(_bf16)
    msg6 = jnp.dot(v, bdmsg_ref[...],
                   preferred_element_type=_f32).astype(_bf16)

    attk = sel(attk6, et, _NE, 0)
    msg = sel(msg6, et, _NE, 0)
    m_ref[...] = msg.astype(_f32)

    prod = qt_ref[...].astype(_bf16) * attk
    att8 = jnp.dot(prod, s_ref[...], preferred_element_type=_f32)
    mu_e = rowsel(mu_ref, et, _NE)
    att_ref[...] = att8 * mu_e * (1.0 / (_DK ** 0.5))


def kernel(h_s, Q_t, etype, tau_s, tau_t, dt, emb, W_rte, b_rte,
           W_K, b_K, W_V, b_V, W_att, W_msg, mu):
    del tau_t

    wrte_t = W_rte.T.astype(_bf16)
    wkv = jnp.concatenate([W_K, W_V], axis=0)
    wkv = jnp.transpose(wkv, (2, 0, 1)).reshape(_IN, 2 * _NT * _OUT)
    wkv = wkv.astype(_bf16)
    def _bd(w):
        b = jnp.zeros((_NE, _OUT, _OUT), _f32)
        wt = jnp.transpose(w, (0, 2, 1))
        for h in range(_H):
            b = b.at[:, h * _DK:(h + 1) * _DK, h * _DK:(h + 1) * _DK].set(wt)
        return jnp.transpose(b, (1, 0, 2)).reshape(_OUT, _NE * _OUT).astype(_bf16)
    bdatt = _bd(W_att)
    bdmsg = _bd(W_msg)
    seg = (jax.lax.broadcasted_iota(jnp.int32, (_OUT, _H), 0) // _DK ==
           jax.lax.broadcasted_iota(jnp.int32, (_OUT, _H), 1)).astype(_bf16)

    dt3 = dt.astype(jnp.int32).reshape(_G, _B, 1)
    tau3 = tau_s.astype(jnp.int32).reshape(_G, _B, 1)
    et3 = etype.astype(jnp.int32).reshape(_G, _B, 1)

    q2 = Q_t.reshape(_E, _IN)

    idx_spec = pl.BlockSpec((1, _B, 1), lambda i: (i, 0, 0))
    row_spec = pl.BlockSpec((_B, _IN), lambda i: (i, 0))

    def w_spec(shape):
        return pl.BlockSpec(shape, lambda i: tuple(0 for _ in shape))

    att, m = pl.pallas_call(
        _body,
        grid=(_G,),
        in_specs=[idx_spec, idx_spec, idx_spec, row_spec, row_spec,
                  w_spec((_ML, _IN)), w_spec((_IN, _IN)), w_spec((_IN,)),
                  w_spec((_IN, 2 * _NT * _OUT)),
                  w_spec((_NT, _OUT)), w_spec((_NT, _OUT)),
                  w_spec((_OUT, _NE * _OUT)), w_spec((_OUT, _NE * _OUT)),
                  w_spec((_OUT, _H)), w_spec((_NE, _H))],
        out_specs=[pl.BlockSpec((_B, _H), lambda i: (i, 0)),
                   pl.BlockSpec((_B, _OUT), lambda i: (i, 0))],
        out_shape=[jax.ShapeDtypeStruct((_E, _H), _f32),
                   jax.ShapeDtypeStruct((_E, _OUT), _f32)],
    )(dt3, tau3, et3, h_s, q2,
      emb, wrte_t, b_rte, wkv, b_K, b_V, bdatt, bdmsg, seg, mu)

    return att, m.reshape(_E, _H, _DK)
